# Initial kernel scaffold; baseline (speedup 1.0000x reference)
import functools

import jax
import jax.numpy as jnp
from jax import lax
from jax.experimental import pallas as pl
from jax.experimental.pallas import tpu as pltpu
from jax.experimental.pallas import tpu_sc as plsc

NN = 50000
NE = 800000
NG = 64
W_S = 16
W_B = 112

NC, NS, LANES = 2, 16, 16
NW = NC * NS

KB = 4
BKW = NN // KB
EPT = NE // NW
EPT_AL = EPT + 16
VPAD = 25600
ACC_B = 12528
CHB = 1024
HALF = NN // 2
ACC_A = 25024
EA = NE // NS
CHA = 2048

_MESH = plsc.VectorSubcoreMesh(
    core_axis_name="c", subcore_axis_name="s", num_cores=NC, num_subcores=NS)

_NEG_INF = float("-inf")


def _lane():
  return lax.iota(jnp.int32, LANES)


@functools.partial(
    pl.kernel,
    out_type=(
        jax.ShapeDtypeStruct((NW, KB, VPAD), jnp.int32),
        jax.ShapeDtypeStruct((NW, KB, VPAD), jnp.int32),
        jax.ShapeDtypeStruct((NW, LANES), jnp.int32),
    ),
    mesh=_MESH,
    scratch_types=[
        pltpu.VMEM((EPT_AL,), jnp.int32),
        pltpu.VMEM((EPT_AL,), jnp.int32),
        pltpu.VMEM((VPAD,), jnp.int32),
        pltpu.VMEM((VPAD,), jnp.int32),
        pltpu.VMEM((LANES,), jnp.int32),
    ],
)
def _bucket_edges(src_hbm, dst_hbm, bsrc_hbm, bdst_hbm, cnt_hbm,
                  src_v, dst_v, osrc_v, odst_v, cnt_v):
  c = lax.axis_index("c")
  s = lax.axis_index("s")
  t = s * NC + c
  base = t * EPT
  pltpu.sync_copy(src_hbm.at[pl.ds(base, EPT)], src_v.at[pl.ds(0, EPT)])
  pltpu.sync_copy(dst_hbm.at[pl.ds(base, EPT)], dst_v.at[pl.ds(0, EPT)])
  lane = _lane()
  nv = (EPT + LANES - 1) // LANES
  counts = jnp.zeros((LANES,), jnp.int32)
  for k in range(KB):
    lo = k * BKW

    def body(i, cur, lo=lo):
      d = dst_v[pl.ds(i * LANES, LANES)]
      sv = src_v[pl.ds(i * LANES, LANES)]
      valid = (i * LANES + lane) < EPT
      m = valid & (d >= lo) & (d < lo + BKW)
      plsc.store_compressed(osrc_v.at[pl.ds(cur, LANES)], sv, m)
      plsc.store_compressed(odst_v.at[pl.ds(cur, LANES)], d - lo, m)
      return cur + jnp.max(plsc.all_reduce_population_count(m))

    cur = lax.fori_loop(0, nv, body, jnp.int32(0))
    counts = jnp.where(lane == k, cur, counts)
    pltpu.sync_copy(osrc_v, bsrc_hbm.at[t, k])
    pltpu.sync_copy(odst_v, bdst_hbm.at[t, k])
  cnt_v[...] = counts
  pltpu.sync_copy(cnt_v, cnt_hbm.at[t])


@functools.partial(
    pl.kernel,
    out_type=jax.ShapeDtypeStruct((NN, W_S), jnp.float32),
    mesh=_MESH,
    scratch_types=[
        pltpu.VMEM_SHARED((NN, W_S), jnp.float32),
        pltpu.VMEM_SHARED((ACC_A, W_S), jnp.float32),
        pltpu.VMEM((CHA, W_S), jnp.float32),
        pltpu.VMEM((CHA,), jnp.int32),
        pltpu.VMEM((CHA,), jnp.int32),
        pltpu.VMEM((CHA // 128, 128), jnp.int32),
        pltpu.VMEM((CHA // 128, 128), jnp.int32),
        pltpu.SemaphoreType.DMA,
        pltpu.SemaphoreType.DMA,
    ],
)
def _aggr_small(table_hbm, src_hbm, dst_hbm, out_hbm,
                tab_sh, acc_sh, rows_v, sflat_v, dflat_v, sidx_v, didx_v,
                gsem, ssem):
  c = lax.axis_index("c")
  s = lax.axis_index("s")
  lane = _lane()
  pltpu.sync_copy(table_hbm.at[pl.ds(s * (NN // NS), NN // NS)],
                  tab_sh.at[pl.ds(s * (NN // NS), NN // NS)])
  zrows = ACC_A // NS

  def zb(j, _):
    r = j // (W_S // LANES)
    m = j % (W_S // LANES)
    rows_v[r, pl.ds(m * LANES, LANES)] = jnp.zeros((LANES,), jnp.float32)
    return 0

  lax.fori_loop(0, zrows * (W_S // LANES), zb, 0)
  pltpu.sync_copy(rows_v.at[pl.ds(0, zrows)],
                  acc_sh.at[pl.ds(s * zrows, zrows)])
  plsc.subcore_barrier()

  base_e = s * EA
  nchunks = (EA + CHA - 1) // CHA
  for q in range(nchunks):
    off = q * CHA
    vsz = min(CHA, EA - off)
    pltpu.sync_copy(src_hbm.at[pl.ds(base_e + off, vsz)],
                    sflat_v.at[pl.ds(0, vsz)])
    pltpu.sync_copy(dst_hbm.at[pl.ds(base_e + off, vsz)],
                    dflat_v.at[pl.ds(0, vsz)])

    def pb(i, _, vsz=vsz):
      r = i // (128 // LANES)
      l = i % (128 // LANES)
      gidx = i * LANES + lane
      m = gidx < vsz
      d = dflat_v[pl.ds(i * LANES, LANES)]
      sv = sflat_v[pl.ds(i * LANES, LANES)]
      loc = d - c * HALF
      inb = m & (loc >= 0) & (loc < HALF)
      didx_v[r, pl.ds(l * LANES, LANES)] = jnp.where(inb, loc, HALF + lane)
      sidx_v[r, pl.ds(l * LANES, LANES)] = jnp.where(m, sv, gidx)
      return 0

    lax.fori_loop(0, CHA // LANES, pb, 0)
    copies = [pltpu.async_copy(tab_sh.at[sidx_v.at[r]],
                               rows_v.at[pl.ds(r * 128, 128)], gsem)
              for r in range(CHA // 128)]
    for cp in copies:
      cp.wait()
    copies = [pltpu.async_copy(rows_v.at[pl.ds(r * 128, 128)],
                               acc_sh.at[didx_v.at[r]], ssem, add=True)
              for r in range(CHA // 128)]
    for cp in copies:
      cp.wait()

  plsc.subcore_barrier()
  osz = 1568
  otail = HALF - (NS - 1) * osz

  @pl.when(s < NS - 1)
  def _():
    pltpu.sync_copy(acc_sh.at[pl.ds(s * osz, osz)],
                    out_hbm.at[pl.ds(c * HALF + s * osz, osz)])

  @pl.when(s == NS - 1)
  def _():
    pltpu.sync_copy(acc_sh.at[pl.ds((NS - 1) * osz, otail)],
                    out_hbm.at[pl.ds(c * HALF + (NS - 1) * osz, otail)])


@functools.partial(
    pl.kernel,
    out_type=jax.ShapeDtypeStruct((NN, W_B), jnp.float32),
    mesh=_MESH,
    scratch_types=[
        pltpu.VMEM_SHARED((ACC_B, W_B), jnp.float32),
        pltpu.VMEM((CHB, W_B), jnp.float32),
        pltpu.VMEM((CHB // 128, 128), jnp.int32),
        pltpu.VMEM((CHB // 128, 128), jnp.int32),
        pltpu.VMEM((LANES,), jnp.int32),
        pltpu.SemaphoreType.DMA,
        pltpu.SemaphoreType.DMA,
    ],
)
def _aggr_big(table_hbm, bsrc_hbm, bdst_hbm, cnt_hbm, out_hbm,
              acc_sh, rows_v, sidx_v, didx_v, cnt_v, gsem, ssem):
  c = lax.axis_index("c")
  s = lax.axis_index("s")
  lane = _lane()
  zrows = ACC_B // NS
  for kk in range(KB // NC):
    k = c * (KB // NC) + kk

    def zb(j, _):
      r = j // (W_B // LANES)
      m = j % (W_B // LANES)
      rows_v[r, pl.ds(m * LANES, LANES)] = jnp.zeros((LANES,), jnp.float32)
      return 0

    lax.fori_loop(0, zrows * (W_B // LANES), zb, 0)
    pltpu.sync_copy(rows_v.at[pl.ds(0, zrows)],
                    acc_sh.at[pl.ds(s * zrows, zrows)])
    plsc.subcore_barrier()

    for j in range(NW // NS):
      tstrip = s + j * NS
      pltpu.sync_copy(cnt_hbm.at[tstrip], cnt_v)
      n = jnp.max(jnp.where(lane == k, cnt_v[...], 0))
      nch = (n + CHB - 1) // CHB

      def chunk(q, _, tstrip=tstrip, n=n):
        off128 = q * (CHB // 128)
        pltpu.sync_copy(bsrc_hbm.at[tstrip, k, pl.ds(off128, CHB // 128)],
                        sidx_v)
        pltpu.sync_copy(bdst_hbm.at[tstrip, k, pl.ds(off128, CHB // 128)],
                        didx_v)
        valid = n - q * CHB

        def pb(i, _):
          r = i // (128 // LANES)
          l = i % (128 // LANES)
          gidx = i * LANES + lane
          m = gidx < valid
          d = didx_v[r, pl.ds(l * LANES, LANES)]
          sv = sidx_v[r, pl.ds(l * LANES, LANES)]
          didx_v[r, pl.ds(l * LANES, LANES)] = jnp.where(m, d, BKW + lane)
          sidx_v[r, pl.ds(l * LANES, LANES)] = jnp.where(m, sv, gidx)
          return 0

        lax.fori_loop(0, CHB // LANES, pb, 0)
        copies = [pltpu.async_copy(table_hbm.at[sidx_v.at[r]],
                                   rows_v.at[pl.ds(r * 128, 128)], gsem)
                  for r in range(CHB // 128)]
        for cp in copies:
          cp.wait()
        copies = [pltpu.async_copy(rows_v.at[pl.ds(r * 128, 128)],
                                   acc_sh.at[didx_v.at[r]], ssem, add=True)
                  for r in range(CHB // 128)]
        for cp in copies:
          cp.wait()
        return 0

      lax.fori_loop(0, nch, chunk, 0)

    plsc.subcore_barrier()
    osz = 784
    otail = BKW - (NS - 1) * osz

    @pl.when(s < NS - 1)
    def _():
      pltpu.sync_copy(acc_sh.at[pl.ds(s * osz, osz)],
                      out_hbm.at[pl.ds(k * BKW + s * osz, osz)])

    @pl.when(s == NS - 1)
    def _():
      pltpu.sync_copy(acc_sh.at[pl.ds((NS - 1) * osz, otail)],
                      out_hbm.at[pl.ds(k * BKW + (NS - 1) * osz, otail)])

    plsc.subcore_barrier()


_PPT = 1568
_NPAD = NW * _PPT
_PCH = 512


@functools.partial(
    pl.kernel,
    out_type=jax.ShapeDtypeStruct((NW, NG * W_B), jnp.float32),
    mesh=_MESH,
    scratch_types=[
        pltpu.VMEM(((NG + LANES) * W_B,), jnp.float32),
        pltpu.VMEM((_PCH * W_B,), jnp.float32),
        pltpu.VMEM((_PPT,), jnp.int32),
    ],
)
def _pool_partials(h_hbm, batch_hbm, part_hbm, part_v, hrows_v, bat_v):
  c = lax.axis_index("c")
  s = lax.axis_index("s")
  lane = _lane()
  t = s * NC + c
  base = t * _PPT

  def ib(j, _):
    part_v[pl.ds(j * LANES, LANES)] = jnp.full((LANES,), _NEG_INF, jnp.float32)
    return 0

  lax.fori_loop(0, (NG + LANES) * W_B // LANES, ib, 0)
  pltpu.sync_copy(batch_hbm.at[pl.ds(base, _PPT)], bat_v)
  off = 0
  while off < _PPT:
    sz = min(_PCH, _PPT - off)
    pltpu.sync_copy(h_hbm.at[pl.ds((base + off) * W_B, sz * W_B)],
                    hrows_v.at[pl.ds(0, sz * W_B)])

    def nb(i, _, off=off):
      node = off + i
      segv = bat_v[pl.ds((node // LANES) * LANES, LANES)]
      seg = jnp.max(jnp.where(lane == node % LANES, segv, 0))
      for m in range(W_B // LANES):
        cur = part_v[pl.ds(seg * W_B + m * LANES, LANES)]
        row = hrows_v[pl.ds(i * W_B + m * LANES, LANES)]
        part_v[pl.ds(seg * W_B + m * LANES, LANES)] = jnp.maximum(cur, row)
      return 0

    lax.fori_loop(0, sz, nb, 0)
    off += sz
  pltpu.sync_copy(part_v.at[pl.ds(0, NG * W_B)], part_hbm.at[t])


_ROWS = 1000


def _dense_body(scale_ref, h_ref, a_ref, w_ref, b_ref, o_ref):
  t = scale_ref[0, 0] * h_ref[...] + a_ref[...]
  z = jnp.dot(t, w_ref[...], preferred_element_type=jnp.float32) + b_ref[...]
  nrm = jnp.sqrt(jnp.sum(z * z, axis=1, keepdims=True))
  z = z / jnp.maximum(nrm, 1e-12)
  o_ref[...] = jnp.maximum(z, 0.0)


@functools.lru_cache(maxsize=None)
def _dense_call(din, dout):
  return pl.pallas_call(
      _dense_body,
      grid=(NN // _ROWS,),
      in_specs=[
          pl.BlockSpec((1, 1), lambda i: (0, 0)),
          pl.BlockSpec((_ROWS, din), lambda i: (i, 0)),
          pl.BlockSpec((_ROWS, din), lambda i: (i, 0)),
          pl.BlockSpec((din, dout), lambda i: (0, 0)),
          pl.BlockSpec((1, dout), lambda i: (0, 0)),
      ],
      out_specs=pl.BlockSpec((_ROWS, dout), lambda i: (i, 0)),
      out_shape=jax.ShapeDtypeStruct((NN, dout), jnp.float32),
  )


def _final_body(part_ref, w_ref, b_ref, o_ref):
  pooled = jnp.max(part_ref[...], axis=0)
  col = lax.broadcasted_iota(jnp.int32, (NG, W_B), 1)
  pooled = jnp.where(col < 100, pooled, 0.0)
  o_ref[...] = (
      jnp.dot(pooled, w_ref[...], preferred_element_type=jnp.float32)
      + b_ref[...])


_final_call = pl.pallas_call(
    _final_body,
    out_shape=jax.ShapeDtypeStruct((NG, 128), jnp.float32),
)


def kernel(x, edge_index, batch, params, out_W, out_b):
  f32 = jnp.float32
  src = edge_index[0]
  dst = edge_index[1]

  bsrc, bdst, cnts = _bucket_edges(src, dst)
  bsrc4 = bsrc.reshape(NW, KB, VPAD // 128, 128)
  bdst4 = bdst.reshape(NW, KB, VPAD // 128, 128)

  def padded(W, b, eps, din_p, dout_p):
    dout, din = W.shape
    Wp = jnp.zeros((din_p, dout_p), f32).at[:din, :dout].set(W.T)
    bp = jnp.zeros((1, dout_p), f32).at[0, :dout].set(b)
    return Wp, bp, (1.0 + eps).reshape(1, 1).astype(f32)

  h = jnp.zeros((NN, W_S), f32).at[:, :1].set(x)
  for i, (W, b, eps) in enumerate(params):
    if i == 0:
      Wp, bp, scl = padded(W, b, eps, W_S, W_S)
      a = _aggr_small(h, src, dst)
      h = _dense_call(W_S, W_S)(scl, h, a, Wp, bp)
    elif i == 1:
      Wp, bp, scl = padded(W, b, eps, W_S, W_B)
      a = _aggr_small(h, src, dst)
      h = _dense_call(W_S, W_B)(scl, h, a, Wp, bp)
    else:
      Wp, bp, scl = padded(W, b, eps, W_B, W_B)
      a = _aggr_big(h, bsrc4, bdst4, cnts)
      h = _dense_call(W_B, W_B)(scl, h, a, Wp, bp)

  hflat = jnp.zeros((_NPAD, W_B), f32).at[:NN].set(h).reshape(-1)
  bpad = jnp.full((_NPAD,), NG, jnp.int32).at[:NN].set(batch)
  part = _pool_partials(hflat, bpad).reshape(NW, NG, W_B)

  WoT = jnp.zeros((W_B, 128), f32).at[:100, 0].set(out_W[0])
  obp = jnp.zeros((1, 128), f32).at[0, 0].set(out_b[0])
  out = _final_call(part, WoT, obp)
  return out[:, 0]


# speedup vs baseline: 10.0346x; 10.0346x over previous
import functools

import jax
import jax.numpy as jnp
from jax import lax
from jax.experimental import pallas as pl
from jax.experimental.pallas import tpu as pltpu
from jax.experimental.pallas import tpu_sc as plsc

NN = 50000
NP = 50048
NE = 800000
NG = 64
W_S = 16
W_B = 112

NC, NS, LANES = 2, 16, 16
NW = NC * NS

KB = 4
BKW = NP // KB
EPT = NE // NW
EPT_AL = EPT + 16
VPAD = 25600
ACC_B = 12544
CHB = 256
HALF = NP // 2
ACC_A = 25088
EA = NE // NS
CHA = 2048

_MESH = plsc.VectorSubcoreMesh(
    core_axis_name="c", subcore_axis_name="s", num_cores=NC, num_subcores=NS)

_NEG_INF = float("-inf")


def _lane():
  return lax.iota(jnp.int32, LANES)


@functools.partial(
    pl.kernel,
    out_type=(
        jax.ShapeDtypeStruct((NW * KB * VPAD,), jnp.int32),
        jax.ShapeDtypeStruct((NW * LANES,), jnp.int32),
    ),
    mesh=_MESH,
    compiler_params=pltpu.CompilerParams(use_tc_tiling_on_sc=False, needs_layout_passes=False),
    scratch_types=[
        pltpu.VMEM((EPT_AL,), jnp.int32),
        pltpu.VMEM((EPT_AL,), jnp.int32),
        pltpu.VMEM((VPAD,), jnp.int32),
        pltpu.VMEM((LANES,), jnp.int32),
    ],
)
def _bucket_edges(src_hbm, dst_hbm, bpack_hbm, cnt_hbm,
                  src_v, dst_v, opack_v, cnt_v):
  c = lax.axis_index("c")
  s = lax.axis_index("s")
  t = s * NC + c
  base = t * EPT
  pltpu.sync_copy(src_hbm.at[pl.ds(base, EPT)], src_v.at[pl.ds(0, EPT)])
  pltpu.sync_copy(dst_hbm.at[pl.ds(base, EPT)], dst_v.at[pl.ds(0, EPT)])
  lane = _lane()
  nv = (EPT + LANES - 1) // LANES
  counts = jnp.zeros((LANES,), jnp.int32)
  for k in range(KB):
    lo = k * BKW

    def body(i, cur, lo=lo):
      d = dst_v[pl.ds(i * LANES, LANES)]
      sv = src_v[pl.ds(i * LANES, LANES)]
      valid = (i * LANES + lane) < EPT
      m = valid & (d >= lo) & (d < lo + BKW)
      packed = (sv << 14) | (d - lo)
      comp = plsc.sort_key_val(packed, packed, mask=m)[0]
      opack_v[pl.ds(cur, LANES)] = comp
      return cur + jnp.max(plsc.all_reduce_population_count(m))

    cur = lax.fori_loop(0, nv, body, jnp.int32(0))
    counts = jnp.where(lane == k, cur, counts)
    pltpu.sync_copy(opack_v, bpack_hbm.at[pl.ds((t * KB + k) * VPAD, VPAD)])
  cnt_v[...] = counts
  pltpu.sync_copy(cnt_v, cnt_hbm.at[pl.ds(t * LANES, LANES)])


@functools.partial(
    pl.kernel,
    out_type=jax.ShapeDtypeStruct((NP, W_S), jnp.float32),
    mesh=_MESH,
    compiler_params=pltpu.CompilerParams(use_tc_tiling_on_sc=False, needs_layout_passes=False),
    scratch_types=[
        pltpu.VMEM_SHARED((NP, W_S), jnp.float32),
        pltpu.VMEM_SHARED((ACC_A, W_S), jnp.float32),
        pltpu.VMEM((CHA, W_S), jnp.float32),
        pltpu.VMEM((CHA,), jnp.int32),
        pltpu.VMEM((CHA,), jnp.int32),
        pltpu.VMEM((CHA // 128, 128), jnp.int32),
        pltpu.VMEM((CHA // 128, 128), jnp.int32),
        pltpu.SemaphoreType.DMA,
        pltpu.SemaphoreType.DMA,
    ],
)
def _aggr_small(table_hbm, src_hbm, dst_hbm, out_hbm,
                tab_sh, acc_sh, rows_v, sflat_v, dflat_v, sidx_v, didx_v,
                gsem, ssem):
  c = lax.axis_index("c")
  s = lax.axis_index("s")
  lane = _lane()
  ssz = NP // NS
  pltpu.sync_copy(table_hbm.at[pl.ds(s * ssz, ssz)],
                  tab_sh.at[pl.ds(s * ssz, ssz)])
  zrows = ACC_A // NS

  def zb(j, _):
    rows_v[j, pl.ds(0, LANES)] = jnp.zeros((LANES,), jnp.float32)
    return 0

  lax.fori_loop(0, zrows, zb, 0)
  pltpu.sync_copy(rows_v.at[pl.ds(0, zrows)],
                  acc_sh.at[pl.ds(s * zrows, zrows)])
  plsc.subcore_barrier()

  base_e = s * EA
  nchunks = (EA + CHA - 1) // CHA
  for q in range(nchunks):
    off = q * CHA
    vsz = min(CHA, EA - off)
    pltpu.sync_copy(src_hbm.at[pl.ds(base_e + off, vsz)],
                    sflat_v.at[pl.ds(0, vsz)])
    pltpu.sync_copy(dst_hbm.at[pl.ds(base_e + off, vsz)],
                    dflat_v.at[pl.ds(0, vsz)])

    def pb(i, _, vsz=vsz):
      r = i // (128 // LANES)
      l = i % (128 // LANES)
      gidx = i * LANES + lane
      m = gidx < vsz
      d = dflat_v[pl.ds(i * LANES, LANES)]
      sv = sflat_v[pl.ds(i * LANES, LANES)]
      loc = d - c * HALF
      inb = m & (loc >= 0) & (loc < HALF)
      didx_v[r, pl.ds(l * LANES, LANES)] = jnp.where(inb, loc, HALF + lane)
      sidx_v[r, pl.ds(l * LANES, LANES)] = jnp.where(m, sv, gidx)
      return 0

    lax.fori_loop(0, CHA // LANES, pb, 0)
    copies = [pltpu.async_copy(tab_sh.at[sidx_v.at[r]],
                               rows_v.at[pl.ds(r * 128, 128)], gsem)
              for r in range(CHA // 128)]
    for cp in copies:
      cp.wait()
    copies = [pltpu.async_copy(rows_v.at[pl.ds(r * 128, 128)],
                               acc_sh.at[didx_v.at[r]], ssem, add=True)
              for r in range(CHA // 128)]
    for cp in copies:
      cp.wait()

  plsc.subcore_barrier()
  osz = 1568
  otail = HALF - (NS - 1) * osz

  @pl.when(s < NS - 1)
  def _():
    pltpu.sync_copy(acc_sh.at[pl.ds(s * osz, osz)],
                    out_hbm.at[pl.ds(c * HALF + s * osz, osz)])

  @pl.when(s == NS - 1)
  def _():
    pltpu.sync_copy(acc_sh.at[pl.ds((NS - 1) * osz, otail)],
                    out_hbm.at[pl.ds(c * HALF + (NS - 1) * osz, otail)])


@functools.partial(
    pl.kernel,
    out_type=jax.ShapeDtypeStruct((NP, W_B), jnp.float32),
    mesh=_MESH,
    compiler_params=pltpu.CompilerParams(use_tc_tiling_on_sc=False, needs_layout_passes=False),
    scratch_types=[
        pltpu.VMEM_SHARED((ACC_B, W_B), jnp.float32),
        pltpu.VMEM((CHB, W_B), jnp.float32),
        pltpu.VMEM((CHB,), jnp.int32),
        pltpu.VMEM((CHB // 128, 128), jnp.int32),
        pltpu.VMEM((CHB // 128, 128), jnp.int32),
        pltpu.VMEM((LANES,), jnp.int32),
        pltpu.SemaphoreType.DMA,
        pltpu.SemaphoreType.DMA,
    ],
)
def _aggr_big(table_hbm, bpack_hbm, cnt_hbm, out_hbm,
              acc_sh, rows_v, pflat_v, sidx_v, didx_v, cnt_v,
              gsem, ssem):
  c = lax.axis_index("c")
  s = lax.axis_index("s")
  lane = _lane()
  zrows = ACC_B // NS
  for kk in range(KB // NC):
    k = c * (KB // NC) + kk

    def zb(j, _):
      for m in range(W_B // LANES):
        rows_v[j, pl.ds(m * LANES, LANES)] = jnp.zeros((LANES,), jnp.float32)
      return 0

    lax.fori_loop(0, CHB, zb, 0)
    zoff = 0
    while zoff < zrows:
      zsz = min(CHB, zrows - zoff)
      pltpu.sync_copy(rows_v.at[pl.ds(0, zsz)],
                      acc_sh.at[pl.ds(s * zrows + zoff, zsz)])
      zoff += zsz
    plsc.subcore_barrier()

    for j in range(NW // NS):
      tstrip = s + j * NS
      pltpu.sync_copy(cnt_hbm.at[pl.ds(tstrip * LANES, LANES)], cnt_v)
      n = jnp.max(jnp.where(lane == k, cnt_v[...], 0))
      nch = (n + CHB - 1) // CHB

      def chunk(q, _, tstrip=tstrip, n=n, k=k):
        off = (tstrip * KB + k) * VPAD + q * CHB
        pltpu.sync_copy(bpack_hbm.at[pl.ds(off, CHB)], pflat_v)
        valid = n - q * CHB

        def pb(i, _):
          r = i // (128 // LANES)
          l = i % (128 // LANES)
          gidx = i * LANES + lane
          m = gidx < valid
          p = pflat_v[pl.ds(i * LANES, LANES)]
          didx_v[r, pl.ds(l * LANES, LANES)] = jnp.where(
              m, p & 16383, BKW + lane)
          sidx_v[r, pl.ds(l * LANES, LANES)] = jnp.where(m, p >> 14, gidx)
          return 0

        lax.fori_loop(0, CHB // LANES, pb, 0)
        copies = [pltpu.async_copy(table_hbm.at[sidx_v.at[r]],
                                   rows_v.at[pl.ds(r * 128, 128)], gsem)
                  for r in range(CHB // 128)]
        for cp in copies:
          cp.wait()
        copies = [pltpu.async_copy(rows_v.at[pl.ds(r * 128, 128)],
                                   acc_sh.at[didx_v.at[r]], ssem, add=True)
                  for r in range(CHB // 128)]
        for cp in copies:
          cp.wait()
        return 0

      lax.fori_loop(0, nch, chunk, 0)

    plsc.subcore_barrier()
    osz = 784
    otail = BKW - (NS - 1) * osz

    @pl.when(s < NS - 1)
    def _():
      pltpu.sync_copy(acc_sh.at[pl.ds(s * osz, osz)],
                      out_hbm.at[pl.ds(k * BKW + s * osz, osz)])

    @pl.when(s == NS - 1)
    def _():
      pltpu.sync_copy(acc_sh.at[pl.ds((NS - 1) * osz, otail)],
                      out_hbm.at[pl.ds(k * BKW + (NS - 1) * osz, otail)])

    plsc.subcore_barrier()


_PPT = 1568
_NPAD = NW * _PPT
_PCH = 512


@functools.partial(
    pl.kernel,
    out_type=jax.ShapeDtypeStruct((NW, NG, W_B), jnp.float32),
    mesh=_MESH,
    compiler_params=pltpu.CompilerParams(use_tc_tiling_on_sc=False, needs_layout_passes=False),
    scratch_types=[
        pltpu.VMEM(((NG + LANES) * W_B,), jnp.float32),
        pltpu.VMEM((NG, W_B), jnp.float32),
        pltpu.VMEM((_PCH * W_B,), jnp.float32),
        pltpu.VMEM((_PPT,), jnp.int32),
    ],
)
def _pool_partials(h_hbm, batch_hbm, part_hbm, part_v, part2_v, hrows_v,
                   bat_v):
  c = lax.axis_index("c")
  s = lax.axis_index("s")
  lane = _lane()
  t = s * NC + c
  base = t * _PPT

  def ib(j, _):
    part_v[pl.ds(j * LANES, LANES)] = jnp.full((LANES,), _NEG_INF, jnp.float32)
    return 0

  lax.fori_loop(0, (NG + LANES) * W_B // LANES, ib, 0)
  pltpu.sync_copy(batch_hbm.at[pl.ds(base, _PPT)], bat_v)
  off = 0
  while off < _PPT:
    sz = min(_PCH, _PPT - off)
    pltpu.sync_copy(h_hbm.at[pl.ds((base + off) * W_B, sz * W_B)],
                    hrows_v.at[pl.ds(0, sz * W_B)])

    def nb(i, _, off=off):
      node = off + i
      segv = bat_v[pl.ds((node // LANES) * LANES, LANES)]
      seg = jnp.max(jnp.where(lane == node % LANES, segv, 0))
      for m in range(W_B // LANES):
        cur = part_v[pl.ds(seg * W_B + m * LANES, LANES)]
        row = hrows_v[pl.ds(i * W_B + m * LANES, LANES)]
        part_v[pl.ds(seg * W_B + m * LANES, LANES)] = jnp.maximum(cur, row)
      return 0

    lax.fori_loop(0, sz, nb, 0)
    off += sz
  for g in range(NG):
    for m in range(W_B // LANES):
      part2_v[g, pl.ds(m * LANES, LANES)] = part_v[
          pl.ds(g * W_B + m * LANES, LANES)]
  pltpu.sync_copy(part2_v, part_hbm.at[t])


_ROWS = 3128


def _dense_body(scale_ref, h_ref, a_ref, w_ref, b_ref, o_ref):
  t = scale_ref[0, 0] * h_ref[...] + a_ref[...]
  z = jnp.dot(t, w_ref[...], preferred_element_type=jnp.float32) + b_ref[...]
  nrm = jnp.sqrt(jnp.sum(z * z, axis=1, keepdims=True))
  z = z / jnp.maximum(nrm, 1e-12)
  o_ref[...] = jnp.maximum(z, 0.0)


@functools.lru_cache(maxsize=None)
def _dense_call(din, dout):
  return pl.pallas_call(
      _dense_body,
      grid=(NP // _ROWS,),
      in_specs=[
          pl.BlockSpec((1, 1), lambda i: (0, 0)),
          pl.BlockSpec((_ROWS, din), lambda i: (i, 0)),
          pl.BlockSpec((_ROWS, din), lambda i: (i, 0)),
          pl.BlockSpec((din, dout), lambda i: (0, 0)),
          pl.BlockSpec((1, dout), lambda i: (0, 0)),
      ],
      out_specs=pl.BlockSpec((_ROWS, dout), lambda i: (i, 0)),
      out_shape=jax.ShapeDtypeStruct((NP, dout), jnp.float32),
  )


def _final_body(part_ref, w_ref, b_ref, o_ref):
  pooled = jnp.max(part_ref[...], axis=0)
  col = lax.broadcasted_iota(jnp.int32, (NG, W_B), 1)
  pooled = jnp.where(col < 100, pooled, 0.0)
  o_ref[...] = (
      jnp.dot(pooled, w_ref[...], preferred_element_type=jnp.float32)
      + b_ref[...])


_final_call = pl.pallas_call(
    _final_body,
    out_shape=jax.ShapeDtypeStruct((NG, 128), jnp.float32),
)


def kernel(x, edge_index, batch, params, out_W, out_b):
  f32 = jnp.float32
  src = edge_index[0]
  dst = edge_index[1]

  bpack, cnts = _bucket_edges(src, dst)

  def padded(W, b, eps, din_p, dout_p):
    dout, din = W.shape
    Wp = jnp.zeros((din_p, dout_p), f32).at[:din, :dout].set(W.T)
    bp = jnp.zeros((1, dout_p), f32).at[0, :dout].set(b)
    return Wp, bp, (1.0 + eps).reshape(1, 1).astype(f32)

  h = jnp.zeros((NP, W_S), f32).at[:NN, :1].set(x)
  for i, (W, b, eps) in enumerate(params):
    if i == 0:
      Wp, bp, scl = padded(W, b, eps, W_S, W_S)
      a = _aggr_small(h, src, dst)
      h = _dense_call(W_S, W_S)(scl, h, a, Wp, bp)
    elif i == 1:
      Wp, bp, scl = padded(W, b, eps, W_S, W_B)
      a = _aggr_small(h, src, dst)
      h = _dense_call(W_S, W_B)(scl, h, a, Wp, bp)
    else:
      Wp, bp, scl = padded(W, b, eps, W_B, W_B)
      a = _aggr_big(h, bpack, cnts)
      h = _dense_call(W_B, W_B)(scl, h, a, Wp, bp)

  hflat = jnp.zeros((_NPAD, W_B), f32).at[:NP].set(h).reshape(-1)
  bpad = jnp.full((_NPAD,), NG, jnp.int32).at[:NN].set(batch)
  part = _pool_partials(hflat, bpad)

  WoT = jnp.zeros((W_B, 128), f32).at[:100, 0].set(out_W[0])
  obp = jnp.zeros((1, 128), f32).at[0, 0].set(out_b[0])
  out = _final_call(part, WoT, obp)
  return out[:, 0]


# speedup vs baseline: 11.9327x; 1.1892x over previous
import functools

import jax
import jax.numpy as jnp
from jax import lax
from jax.experimental import pallas as pl
from jax.experimental.pallas import tpu as pltpu
from jax.experimental.pallas import tpu_sc as plsc

NN = 50000
NP = 50048
NE = 800000
NG = 64
W_S = 16
W_B = 112

NC, NS, LANES = 2, 16, 16
NW = NC * NS

KB = 8
BKW = NP // KB
EPT = NE // NW
EPT_AL = EPT + 16
VPAD = 25088
ACC_B = 6272
BLK = 512
SUB = 128
HALF = NP // 2
ACC_A = 25088
EA = NE // NS
CHA = 2048

_MESH = plsc.VectorSubcoreMesh(
    core_axis_name="c", subcore_axis_name="s", num_cores=NC, num_subcores=NS)

_NEG_INF = float("-inf")


def _lane():
  return lax.iota(jnp.int32, LANES)


@functools.partial(
    pl.kernel,
    out_type=(
        jax.ShapeDtypeStruct((NW * KB * VPAD,), jnp.int32),
        jax.ShapeDtypeStruct((NW * LANES,), jnp.int32),
    ),
    mesh=_MESH,
    compiler_params=pltpu.CompilerParams(use_tc_tiling_on_sc=False, needs_layout_passes=False),
    scratch_types=[
        pltpu.VMEM((EPT_AL,), jnp.int32),
        pltpu.VMEM((EPT_AL,), jnp.int32),
        pltpu.VMEM((VPAD,), jnp.int32),
        pltpu.VMEM((VPAD,), jnp.int32),
        pltpu.VMEM((LANES,), jnp.int32),
    ],
)
def _bucket_edges(src_hbm, dst_hbm, bpack_hbm, cnt_hbm,
                  src_v, dst_v, opack_v, opack1_v, cnt_v):
  c = lax.axis_index("c")
  s = lax.axis_index("s")
  t = s * NC + c
  base = t * EPT
  pltpu.sync_copy(src_hbm.at[pl.ds(base, EPT)], src_v.at[pl.ds(0, EPT)])
  pltpu.sync_copy(dst_hbm.at[pl.ds(base, EPT)], dst_v.at[pl.ds(0, EPT)])
  lane = _lane()
  nv = (EPT + LANES - 1) // LANES
  counts = jnp.zeros((LANES,), jnp.int32)
  for pp in range(KB // 2):
    k0 = 2 * pp
    lo0 = k0 * BKW
    lo1 = lo0 + BKW

    def body(i, curs, lo0=lo0, lo1=lo1):
      cur0, cur1 = curs
      d = dst_v[pl.ds(i * LANES, LANES)]
      sv = src_v[pl.ds(i * LANES, LANES)]
      valid = (i * LANES + lane) < EPT
      m0 = valid & (d >= lo0) & (d < lo1)
      m1 = valid & (d >= lo1) & (d < lo1 + BKW)
      p0 = (sv << 14) | (d - lo0)
      p1 = (sv << 14) | (d - lo1)
      c0 = plsc.sort_key_val(p0, p0, mask=m0)[0]
      c1 = plsc.sort_key_val(p1, p1, mask=m1)[0]
      opack_v[pl.ds(cur0, LANES)] = c0
      opack1_v[pl.ds(cur1, LANES)] = c1
      cur0 = cur0 + jnp.max(plsc.all_reduce_population_count(m0))
      cur1 = cur1 + jnp.max(plsc.all_reduce_population_count(m1))
      return cur0, cur1

    cur0, cur1 = lax.fori_loop(0, nv, body, (jnp.int32(0), jnp.int32(0)))
    counts = jnp.where(lane == k0, cur0, counts)
    counts = jnp.where(lane == k0 + 1, cur1, counts)
    pltpu.sync_copy(opack_v, bpack_hbm.at[pl.ds((t * KB + k0) * VPAD, VPAD)])
    pltpu.sync_copy(opack1_v,
                    bpack_hbm.at[pl.ds((t * KB + k0 + 1) * VPAD, VPAD)])
  cnt_v[...] = counts
  pltpu.sync_copy(cnt_v, cnt_hbm.at[pl.ds(t * LANES, LANES)])


@functools.partial(
    pl.kernel,
    out_type=jax.ShapeDtypeStruct((NP, W_S), jnp.float32),
    mesh=_MESH,
    compiler_params=pltpu.CompilerParams(use_tc_tiling_on_sc=False, needs_layout_passes=False),
    scratch_types=[
        pltpu.VMEM_SHARED((NP, W_S), jnp.float32),
        pltpu.VMEM_SHARED((ACC_A, W_S), jnp.float32),
        pltpu.VMEM((CHA, W_S), jnp.float32),
        pltpu.VMEM((CHA,), jnp.int32),
        pltpu.VMEM((CHA,), jnp.int32),
        pltpu.VMEM((CHA // 128, 128), jnp.int32),
        pltpu.VMEM((CHA // 128, 128), jnp.int32),
        pltpu.SemaphoreType.DMA,
        pltpu.SemaphoreType.DMA,
    ],
)
def _aggr_small(table_hbm, src_hbm, dst_hbm, out_hbm,
                tab_sh, acc_sh, rows_v, sflat_v, dflat_v, sidx_v, didx_v,
                gsem, ssem):
  c = lax.axis_index("c")
  s = lax.axis_index("s")
  lane = _lane()
  ssz = NP // NS
  pltpu.sync_copy(table_hbm.at[pl.ds(s * ssz, ssz)],
                  tab_sh.at[pl.ds(s * ssz, ssz)])
  zrows = ACC_A // NS

  def zb(j, _):
    rows_v[j, pl.ds(0, LANES)] = jnp.zeros((LANES,), jnp.float32)
    return 0

  lax.fori_loop(0, zrows, zb, 0)
  pltpu.sync_copy(rows_v.at[pl.ds(0, zrows)],
                  acc_sh.at[pl.ds(s * zrows, zrows)])
  plsc.subcore_barrier()

  base_e = s * EA
  nchunks = (EA + CHA - 1) // CHA
  for q in range(nchunks):
    off = q * CHA
    vsz = min(CHA, EA - off)
    pltpu.sync_copy(src_hbm.at[pl.ds(base_e + off, vsz)],
                    sflat_v.at[pl.ds(0, vsz)])
    pltpu.sync_copy(dst_hbm.at[pl.ds(base_e + off, vsz)],
                    dflat_v.at[pl.ds(0, vsz)])

    def pb(i, _, vsz=vsz):
      r = i // (128 // LANES)
      l = i % (128 // LANES)
      gidx = i * LANES + lane
      m = gidx < vsz
      d = dflat_v[pl.ds(i * LANES, LANES)]
      sv = sflat_v[pl.ds(i * LANES, LANES)]
      loc = d - c * HALF
      inb = m & (loc >= 0) & (loc < HALF)
      didx_v[r, pl.ds(l * LANES, LANES)] = jnp.where(inb, loc, HALF + lane)
      sidx_v[r, pl.ds(l * LANES, LANES)] = jnp.where(m, sv, gidx)
      return 0

    lax.fori_loop(0, CHA // LANES, pb, 0)
    copies = [pltpu.async_copy(tab_sh.at[sidx_v.at[r]],
                               rows_v.at[pl.ds(r * 128, 128)], gsem)
              for r in range(CHA // 128)]
    for cp in copies:
      cp.wait()
    copies = [pltpu.async_copy(rows_v.at[pl.ds(r * 128, 128)],
                               acc_sh.at[didx_v.at[r]], ssem, add=True)
              for r in range(CHA // 128)]
    for cp in copies:
      cp.wait()

  plsc.subcore_barrier()
  osz = 1568
  otail = HALF - (NS - 1) * osz

  @pl.when(s < NS - 1)
  def _():
    pltpu.sync_copy(acc_sh.at[pl.ds(s * osz, osz)],
                    out_hbm.at[pl.ds(c * HALF + s * osz, osz)])

  @pl.when(s == NS - 1)
  def _():
    pltpu.sync_copy(acc_sh.at[pl.ds((NS - 1) * osz, otail)],
                    out_hbm.at[pl.ds(c * HALF + (NS - 1) * osz, otail)])


@functools.partial(
    pl.kernel,
    out_type=jax.ShapeDtypeStruct((NP, W_B), jnp.float32),
    mesh=_MESH,
    compiler_params=pltpu.CompilerParams(use_tc_tiling_on_sc=False, needs_layout_passes=False),
    scratch_types=[
        pltpu.VMEM_SHARED((ACC_B, W_B), jnp.float32),
        pltpu.VMEM((BLK, W_B), jnp.float32),
        pltpu.VMEM((2 * BLK,), jnp.int32),
        pltpu.VMEM((2 * BLK // 128, 128), jnp.int32),
        pltpu.VMEM((2 * BLK // 128, 128), jnp.int32),
        pltpu.VMEM((LANES,), jnp.int32),
        pltpu.SemaphoreType.DMA,
        pltpu.SemaphoreType.DMA,
    ],
)
def _aggr_big(table_hbm, bpack_hbm, cnt_hbm, out_hbm,
              acc_sh, rows_v, pflat_v, sidx_v, didx_v, cnt_v,
              gsem, ssem):
  c = lax.axis_index("c")
  s = lax.axis_index("s")
  lane = _lane()
  zrows = ACC_B // NS
  nring = BLK // SUB

  for kk in range(KB // NC):
    k = c * (KB // NC) + kk

    def zb(j, _):
      for m in range(W_B // LANES):
        rows_v[j, pl.ds(m * LANES, LANES)] = jnp.zeros((LANES,), jnp.float32)
      return 0

    lax.fori_loop(0, zrows, zb, 0)
    pltpu.sync_copy(rows_v.at[pl.ds(0, zrows)],
                    acc_sh.at[pl.ds(s * zrows, zrows)])
    plsc.subcore_barrier()

    for j in range(NW // NS):
      tstrip = s + j * NS
      pltpu.sync_copy(cnt_hbm.at[pl.ds(tstrip * LANES, LANES)], cnt_v)
      n = jnp.max(jnp.where(lane == k, cnt_v[...], 0))
      nblk = (n + BLK - 1) // BLK

      def block(q, _, tstrip=tstrip, n=n, k=k):
        par = q % 2
        src_off = (tstrip * KB + k) * VPAD + q * BLK
        pltpu.sync_copy(bpack_hbm.at[pl.ds(src_off, BLK)],
                        pflat_v.at[pl.ds(par * BLK, BLK)])
        valid = n - q * BLK

        def pb(i, _):
          r = par * nring + i // (SUB // LANES)
          l = i % (SUB // LANES)
          gidx = i * LANES + lane
          m = gidx < valid
          pck = pflat_v[pl.ds(par * BLK + i * LANES, LANES)]
          didx_v[r, pl.ds(l * LANES, LANES)] = jnp.where(
              m, pck & 16383, BKW + lane)
          sidx_v[r, pl.ds(l * LANES, LANES)] = jnp.where(m, pck >> 14, gidx)
          return 0

        lax.fori_loop(0, BLK // LANES, pb, 0)

        @pl.when(q > 0)
        def _():
          for r in range(nring):
            pltpu.make_async_copy(
                rows_v.at[pl.ds(r * SUB, SUB)],
                acc_sh.at[didx_v.at[r]], ssem).wait()

        ghs = [pltpu.async_copy(table_hbm.at[sidx_v.at[par * nring + r]],
                                rows_v.at[pl.ds(r * SUB, SUB)], gsem)
               for r in range(nring)]
        for r in range(nring):
          ghs[r].wait()
          pltpu.async_copy(rows_v.at[pl.ds(r * SUB, SUB)],
                           acc_sh.at[didx_v.at[par * nring + r]], ssem,
                           add=True)
        return 0

      lax.fori_loop(0, nblk, block, 0)

      @pl.when(nblk > 0)
      def _():
        for r in range(nring):
          pltpu.make_async_copy(
              rows_v.at[pl.ds(r * SUB, SUB)],
              acc_sh.at[didx_v.at[r]], ssem).wait()

    plsc.subcore_barrier()
    osz = 392
    otail = BKW - (NS - 1) * osz

    @pl.when(s < NS - 1)
    def _():
      pltpu.sync_copy(acc_sh.at[pl.ds(s * osz, osz)],
                      out_hbm.at[pl.ds(k * BKW + s * osz, osz)])

    @pl.when(s == NS - 1)
    def _():
      pltpu.sync_copy(acc_sh.at[pl.ds((NS - 1) * osz, otail)],
                      out_hbm.at[pl.ds(k * BKW + (NS - 1) * osz, otail)])

    plsc.subcore_barrier()


_PPT = 1568
_NPAD = NW * _PPT
_PCH = 512


@functools.partial(
    pl.kernel,
    out_type=jax.ShapeDtypeStruct((NW, NG, W_B), jnp.float32),
    mesh=_MESH,
    compiler_params=pltpu.CompilerParams(use_tc_tiling_on_sc=False, needs_layout_passes=False),
    scratch_types=[
        pltpu.VMEM(((NG + LANES) * W_B,), jnp.float32),
        pltpu.VMEM((NG, W_B), jnp.float32),
        pltpu.VMEM((_PCH * W_B,), jnp.float32),
        pltpu.VMEM((_PPT,), jnp.int32),
    ],
)
def _pool_partials(h_hbm, batch_hbm, part_hbm, part_v, part2_v, hrows_v,
                   bat_v):
  c = lax.axis_index("c")
  s = lax.axis_index("s")
  lane = _lane()
  t = s * NC + c
  base = t * _PPT

  def ib(j, _):
    part_v[pl.ds(j * LANES, LANES)] = jnp.full((LANES,), _NEG_INF, jnp.float32)
    return 0

  lax.fori_loop(0, (NG + LANES) * W_B // LANES, ib, 0)
  pltpu.sync_copy(batch_hbm.at[pl.ds(base, _PPT)], bat_v)
  off = 0
  while off < _PPT:
    sz = min(_PCH, _PPT - off)
    pltpu.sync_copy(h_hbm.at[pl.ds((base + off) * W_B, sz * W_B)],
                    hrows_v.at[pl.ds(0, sz * W_B)])

    def nb(i, _, off=off):
      node = off + i
      segv = bat_v[pl.ds((node // LANES) * LANES, LANES)]
      seg = jnp.max(jnp.where(lane == node % LANES, segv, 0))
      for m in range(W_B // LANES):
        cur = part_v[pl.ds(seg * W_B + m * LANES, LANES)]
        row = hrows_v[pl.ds(i * W_B + m * LANES, LANES)]
        part_v[pl.ds(seg * W_B + m * LANES, LANES)] = jnp.maximum(cur, row)
      return 0

    lax.fori_loop(0, sz, nb, 0)
    off += sz
  for g in range(NG):
    for m in range(W_B // LANES):
      part2_v[g, pl.ds(m * LANES, LANES)] = part_v[
          pl.ds(g * W_B + m * LANES, LANES)]
  pltpu.sync_copy(part2_v, part_hbm.at[t])


_ROWS = 3128


def _dense_body(scale_ref, h_ref, a_ref, w_ref, b_ref, o_ref):
  t = scale_ref[0, 0] * h_ref[...] + a_ref[...]
  z = jnp.dot(t, w_ref[...], preferred_element_type=jnp.float32) + b_ref[...]
  nrm = jnp.sqrt(jnp.sum(z * z, axis=1, keepdims=True))
  z = z / jnp.maximum(nrm, 1e-12)
  o_ref[...] = jnp.maximum(z, 0.0)


@functools.lru_cache(maxsize=None)
def _dense_call(din, dout):
  return pl.pallas_call(
      _dense_body,
      grid=(NP // _ROWS,),
      in_specs=[
          pl.BlockSpec((1, 1), lambda i: (0, 0)),
          pl.BlockSpec((_ROWS, din), lambda i: (i, 0)),
          pl.BlockSpec((_ROWS, din), lambda i: (i, 0)),
          pl.BlockSpec((din, dout), lambda i: (0, 0)),
          pl.BlockSpec((1, dout), lambda i: (0, 0)),
      ],
      out_specs=pl.BlockSpec((_ROWS, dout), lambda i: (i, 0)),
      out_shape=jax.ShapeDtypeStruct((NP, dout), jnp.float32),
  )


def _final_body(part_ref, w_ref, b_ref, o_ref):
  pooled = jnp.max(part_ref[...], axis=0)
  col = lax.broadcasted_iota(jnp.int32, (NG, W_B), 1)
  pooled = jnp.where(col < 100, pooled, 0.0)
  o_ref[...] = (
      jnp.dot(pooled, w_ref[...], preferred_element_type=jnp.float32)
      + b_ref[...])


_final_call = pl.pallas_call(
    _final_body,
    out_shape=jax.ShapeDtypeStruct((NG, 128), jnp.float32),
)


def kernel(x, edge_index, batch, params, out_W, out_b):
  f32 = jnp.float32
  src = edge_index[0]
  dst = edge_index[1]

  bpack, cnts = _bucket_edges(src, dst)

  def padded(W, b, eps, din_p, dout_p):
    dout, din = W.shape
    Wp = jnp.zeros((din_p, dout_p), f32).at[:din, :dout].set(W.T)
    bp = jnp.zeros((1, dout_p), f32).at[0, :dout].set(b)
    return Wp, bp, (1.0 + eps).reshape(1, 1).astype(f32)

  h = jnp.zeros((NP, W_S), f32).at[:NN, :1].set(x)
  for i, (W, b, eps) in enumerate(params):
    if i == 0:
      Wp, bp, scl = padded(W, b, eps, W_S, W_S)
      a = _aggr_small(h, src, dst)
      h = _dense_call(W_S, W_S)(scl, h, a, Wp, bp)
    elif i == 1:
      Wp, bp, scl = padded(W, b, eps, W_S, W_B)
      a = _aggr_small(h, src, dst)
      h = _dense_call(W_S, W_B)(scl, h, a, Wp, bp)
    else:
      Wp, bp, scl = padded(W, b, eps, W_B, W_B)
      a = _aggr_big(h, bpack, cnts)
      h = _dense_call(W_B, W_B)(scl, h, a, Wp, bp)

  hflat = jnp.zeros((_NPAD, W_B), f32).at[:NP].set(h).reshape(-1)
  bpad = jnp.full((_NPAD,), NG, jnp.int32).at[:NN].set(batch)
  part = _pool_partials(hflat, bpad)

  WoT = jnp.zeros((W_B, 128), f32).at[:100, 0].set(out_W[0])
  obp = jnp.zeros((1, 128), f32).at[0, 0].set(out_b[0])
  out = _final_call(part, WoT, obp)
  return out[:, 0]
